# baseline (speedup 1.0000x reference)
import functools

import jax
import jax.numpy as jnp
from jax import lax
from jax.experimental import pallas as pl
from jax.experimental.pallas import tpu as pltpu
from jax.experimental.pallas import tpu_sc as plsc

NUM_EMB = 1000000
DIM = 32
BATCH = 16384
N_FIELDS = 26
B_TOTAL = BATCH * N_FIELDS

_info = plsc.get_sparse_core_info()
NC = _info.num_cores
NS = _info.num_subcores
NW = NC * NS
B_PER_W = B_TOTAL // NW
CHUNK = 1024
N_CHUNKS = B_PER_W // CHUNK
NBUF = 3

_mesh = plsc.VectorSubcoreMesh(core_axis_name="c", subcore_axis_name="s")


@functools.partial(
    pl.kernel,
    mesh=_mesh,
    out_type=jax.ShapeDtypeStruct((B_TOTAL, DIM), jnp.float32),
    scratch_types=[
        pltpu.VMEM((N_CHUNKS, CHUNK), jnp.int32),
        [pltpu.VMEM((CHUNK, DIM), jnp.float32) for _ in range(NBUF)],
        [pltpu.SemaphoreType.DMA for _ in range(NBUF)],
        [pltpu.SemaphoreType.DMA for _ in range(NBUF)],
    ],
    compiler_params=pltpu.CompilerParams(use_tc_tiling_on_sc=False),
)
def _emb_gather(idx_hbm, table_hbm, out_hbm, idx_v, rows, sem_g, sem_o):
    wid = lax.axis_index("s") * NC + lax.axis_index("c")
    base = wid * B_PER_W

    pltpu.sync_copy(idx_hbm.at[wid], idx_v)

    def start_gather(i, b):
        pltpu.make_async_copy(table_hbm.at[idx_v.at[i]], rows[b], sem_g[b]).start()

    for i in range(NBUF):
        start_gather(i, i)

    for i in range(N_CHUNKS):
        b = i % NBUF
        off = base + i * CHUNK
        pltpu.make_async_copy(table_hbm.at[idx_v.at[i]], rows[b], sem_g[b]).wait()
        store = pltpu.async_copy(rows[b], out_hbm.at[pl.ds(off, CHUNK)], sem_o[b])
        store.wait()
        if i + NBUF < N_CHUNKS:
            start_gather(i + NBUF, b)


def kernel(indices, weight):
    idx = lax.rem(indices.astype(jnp.int32), jnp.int32(NUM_EMB))
    flat_idx = idx.reshape(NW, N_CHUNKS, CHUNK)
    out = _emb_gather(flat_idx, weight)
    res = out.reshape(BATCH, N_FIELDS, DIM)
    return jnp.where(res == res, res, jnp.float32(0))


# speedup vs baseline: 1.8573x; 1.8573x over previous
import functools

import jax
import jax.numpy as jnp
from jax import lax
from jax.experimental import pallas as pl
from jax.experimental.pallas import tpu as pltpu
from jax.experimental.pallas import tpu_sc as plsc

NUM_EMB = 1000000
DIM = 32
BATCH = 16384
N_FIELDS = 26
B_TOTAL = BATCH * N_FIELDS

_info = plsc.get_sparse_core_info()
NC = _info.num_cores
NS = _info.num_subcores
NW = NC * NS
B_PER_W = B_TOTAL // NW
CHUNK = 1024
N_CHUNKS = B_PER_W // CHUNK
NBUF = 3

_mesh = plsc.VectorSubcoreMesh(core_axis_name="c", subcore_axis_name="s")


@functools.partial(
    pl.kernel,
    mesh=_mesh,
    out_type=jax.ShapeDtypeStruct((B_TOTAL, DIM), jnp.float32),
    scratch_types=[
        pltpu.VMEM((N_CHUNKS, CHUNK), jnp.int32),
        [pltpu.VMEM((CHUNK, DIM), jnp.float32) for _ in range(NBUF)],
        [pltpu.SemaphoreType.DMA for _ in range(NBUF)],
        [pltpu.SemaphoreType.DMA for _ in range(NBUF)],
    ],
    compiler_params=pltpu.CompilerParams(use_tc_tiling_on_sc=False),
)
def _emb_gather(idx_hbm, table_hbm, out_hbm, idx_v, rows, sem_g, sem_o):
    wid = lax.axis_index("s") * NC + lax.axis_index("c")
    base = wid * B_PER_W

    pltpu.sync_copy(idx_hbm.at[wid], idx_v)

    def start_gather(i, b):
        pltpu.make_async_copy(table_hbm.at[idx_v.at[i]], rows[b], sem_g[b]).start()

    for i in range(NBUF):
        start_gather(i, i)

    for i in range(N_CHUNKS):
        b = i % NBUF
        off = base + i * CHUNK
        pltpu.make_async_copy(table_hbm.at[idx_v.at[i]], rows[b], sem_g[b]).wait()
        store = pltpu.async_copy(rows[b], out_hbm.at[pl.ds(off, CHUNK)], sem_o[b])
        store.wait()
        if i + NBUF < N_CHUNKS:
            start_gather(i + NBUF, b)


def kernel(indices, weight):
    flat_idx = indices.reshape(NW, N_CHUNKS, CHUNK).astype(jnp.int32)
    out = _emb_gather(flat_idx, weight)
    return out.reshape(BATCH, N_FIELDS, DIM)
